# baseline (device time: 273227 ns/iter reference)
import jax
import jax.numpy as jnp
from jax import lax
from jax.experimental import pallas as pl
from jax.experimental.pallas import tpu as pltpu

N_DEV = 8
B, SQ, SKV, DM = 2, 512, 512, 768
HQ_PER = 8
DH = 64
DP = HQ_PER * DH
BLK = 64


def kernel(x, Wq, K_ext, V_ext, Wo):
    my = lax.axis_index("i")
    Kh = lax.dynamic_slice_in_dim(K_ext, my * HQ_PER, HQ_PER, axis=2)
    Vh = lax.dynamic_slice_in_dim(V_ext, my * HQ_PER, HQ_PER, axis=2)
    Kt = jnp.transpose(Kh, (0, 2, 1, 3))
    Vt = jnp.transpose(Vh, (0, 2, 1, 3))

    def body(x_ref, wq_ref, k_ref, v_ref, wo_ref, out_ref,
             comm_ref, ctx_ref, send_sems, recv_sems):
        my_pos = lax.axis_index("i")
        left = (my_pos - 1) % N_DEV
        right = (my_pos + 1) % N_DEV

        barrier_sem = pltpu.get_barrier_semaphore()
        for nbr in (left, right):
            pl.semaphore_signal(
                barrier_sem, inc=1,
                device_id=(nbr,), device_id_type=pl.DeviceIdType.MESH,
            )
        pl.semaphore_wait(barrier_sem, 2)

        qb = lax.broadcasted_iota(jnp.int32, (SQ, SKV), 0) // BLK
        kb = lax.broadcasted_iota(jnp.int32, (SQ, SKV), 1) // BLK
        mask = kb <= qb

        for b in range(B):
            q_b = jnp.dot(x_ref[b], wq_ref[...],
                          preferred_element_type=jnp.float32)
            for h in range(HQ_PER):
                q = q_b[:, h * DH:(h + 1) * DH]
                s = lax.dot_general(
                    q, k_ref[b, h], (((1,), (1,)), ((), ())),
                    preferred_element_type=jnp.float32) * 0.125
                s = jnp.where(mask, s, -1e9)
                m = jnp.max(s, axis=-1, keepdims=True)
                w = jnp.exp(s - m)
                w = w / jnp.sum(w, axis=-1, keepdims=True)
                ctx_ref[:, h * DH:(h + 1) * DH] = jnp.dot(
                    w, v_ref[b, h], preferred_element_type=jnp.float32)
            part = jnp.dot(ctx_ref[...], wo_ref[...],
                           preferred_element_type=jnp.float32)
            out_ref[b] = part
            comm_ref[0, b] = part

        for hop in range(N_DEV - 1):
            send_slot = hop % 2
            recv_slot = (hop + 1) % 2
            rdma = pltpu.make_async_remote_copy(
                src_ref=comm_ref.at[send_slot],
                dst_ref=comm_ref.at[recv_slot],
                send_sem=send_sems.at[send_slot],
                recv_sem=recv_sems.at[recv_slot],
                device_id=(right,),
                device_id_type=pl.DeviceIdType.MESH,
            )
            rdma.start()
            rdma.wait()
            for b in range(B):
                out_ref[b] = out_ref[b] + comm_ref[recv_slot, b]

    return pl.pallas_call(
        body,
        out_shape=jax.ShapeDtypeStruct((B, SQ, DM), jnp.float32),
        in_specs=[pl.BlockSpec(memory_space=pltpu.VMEM)] * 5,
        out_specs=pl.BlockSpec(memory_space=pltpu.VMEM),
        scratch_shapes=[
            pltpu.VMEM((2, B, SQ, DM), jnp.float32),
            pltpu.VMEM((SQ, DP), jnp.float32),
            pltpu.SemaphoreType.DMA((2,)),
            pltpu.SemaphoreType.DMA((2,)),
        ],
        compiler_params=pltpu.CompilerParams(collective_id=0),
    )(x, Wq, Kt, Vt, Wo)


# device time: 106887 ns/iter; 2.5562x vs baseline; 2.5562x over previous
import jax
import jax.numpy as jnp
from jax import lax
from jax.experimental import pallas as pl
from jax.experimental.pallas import tpu as pltpu

N_DEV = 8
B, SQ, SKV, DM = 2, 512, 512, 768
HQ_PER = 8
DH = 64
DP = HQ_PER * DH
BLK = 64
ROWS = B * SQ
CHUNK = ROWS // N_DEV


def kernel(x, Wq, K_ext, V_ext, Wo):
    my = lax.axis_index("i")
    Kh = lax.dynamic_slice_in_dim(K_ext, my * HQ_PER, HQ_PER, axis=2)
    Vh = lax.dynamic_slice_in_dim(V_ext, my * HQ_PER, HQ_PER, axis=2)
    Kt = jnp.transpose(Kh, (0, 2, 1, 3))
    Vt = jnp.transpose(Vh, (0, 2, 1, 3))
    xf = x.reshape(ROWS, DM)

    def body(x_ref, wq_ref, k_ref, v_ref, wo_ref, out_ref,
             comm_ref, ctx_ref, send_sems, recv_sems):
        my_pos = lax.axis_index("i")
        left = (my_pos - 1) % N_DEV
        right = (my_pos + 1) % N_DEV

        barrier_sem = pltpu.get_barrier_semaphore()
        for nbr in (left, right):
            pl.semaphore_signal(
                barrier_sem, inc=1,
                device_id=(nbr,), device_id_type=pl.DeviceIdType.MESH,
            )
        pl.semaphore_wait(barrier_sem, 2)

        qb = lax.broadcasted_iota(jnp.int32, (SQ, SKV), 0) // BLK
        kb = lax.broadcasted_iota(jnp.int32, (SQ, SKV), 1) // BLK
        mask = kb <= qb

        q_all = jnp.dot(x_ref[...], wq_ref[...],
                        preferred_element_type=jnp.float32)
        for b in range(B):
            for h in range(HQ_PER):
                q = q_all[b * SQ:(b + 1) * SQ, h * DH:(h + 1) * DH]
                s = lax.dot_general(
                    q, k_ref[b, h], (((1,), (1,)), ((), ())),
                    preferred_element_type=jnp.float32) * 0.125
                s = jnp.where(mask, s, -1e9)
                m = jnp.max(s, axis=-1, keepdims=True)
                w = jnp.exp(s - m)
                w = w / jnp.sum(w, axis=-1, keepdims=True)
                ctx_ref[b * SQ:(b + 1) * SQ, h * DH:(h + 1) * DH] = jnp.dot(
                    w, v_ref[b, h], preferred_element_type=jnp.float32)
        out_ref[...] = jnp.dot(ctx_ref[...], wo_ref[...],
                               preferred_element_type=jnp.float32)

        for t in range(N_DEV - 1):
            sc = (my_pos - t) % N_DEV
            rc = (my_pos - t - 1) % N_DEV
            slot = t % 2
            rdma = pltpu.make_async_remote_copy(
                src_ref=out_ref.at[pl.ds(sc * CHUNK, CHUNK), :],
                dst_ref=comm_ref.at[slot],
                send_sem=send_sems.at[slot],
                recv_sem=recv_sems.at[slot],
                device_id=(right,),
                device_id_type=pl.DeviceIdType.MESH,
            )
            rdma.start()
            rdma.wait()
            out_ref[pl.ds(rc * CHUNK, CHUNK), :] = (
                out_ref[pl.ds(rc * CHUNK, CHUNK), :] + comm_ref[slot])

        for t in range(N_DEV - 1):
            ac = (my_pos + 1 - t) % N_DEV
            slot = t % 2
            rdma = pltpu.make_async_remote_copy(
                src_ref=out_ref.at[pl.ds(ac * CHUNK, CHUNK), :],
                dst_ref=out_ref.at[pl.ds(ac * CHUNK, CHUNK), :],
                send_sem=send_sems.at[slot],
                recv_sem=recv_sems.at[slot],
                device_id=(right,),
                device_id_type=pl.DeviceIdType.MESH,
            )
            rdma.start()
            rdma.wait()

    out2d = pl.pallas_call(
        body,
        out_shape=jax.ShapeDtypeStruct((ROWS, DM), jnp.float32),
        in_specs=[pl.BlockSpec(memory_space=pltpu.VMEM)] * 5,
        out_specs=pl.BlockSpec(memory_space=pltpu.VMEM),
        scratch_shapes=[
            pltpu.VMEM((2, CHUNK, DM), jnp.float32),
            pltpu.VMEM((ROWS, DP), jnp.float32),
            pltpu.SemaphoreType.DMA((2,)),
            pltpu.SemaphoreType.DMA((2,)),
        ],
        compiler_params=pltpu.CompilerParams(collective_id=0),
    )(xf, Wq, Kt, Vt, Wo)
    return out2d.reshape(B, SQ, DM)


# device time: 83262 ns/iter; 3.2815x vs baseline; 1.2837x over previous
import jax
import jax.numpy as jnp
from jax import lax
from jax.experimental import pallas as pl
from jax.experimental.pallas import tpu as pltpu

N_DEV = 8
B, SQ, SKV, DM = 2, 512, 512, 768
HQ_PER = 8
DH = 64
DP = HQ_PER * DH
BLK = 64
ROWS = B * SQ
HALF = ROWS // 2
CHUNK = HALF // N_DEV


def kernel(x, Wq, K_ext, V_ext, Wo):
    my = lax.axis_index("i")
    Kh = lax.dynamic_slice_in_dim(K_ext, my * HQ_PER, HQ_PER, axis=2)
    Vh = lax.dynamic_slice_in_dim(V_ext, my * HQ_PER, HQ_PER, axis=2)
    Kt = jnp.transpose(Kh, (0, 2, 1, 3))
    Vt = jnp.transpose(Vh, (0, 2, 1, 3))
    xf = x.reshape(ROWS, DM)

    def body(x_ref, wq_ref, k_ref, v_ref, wo_ref, out_ref,
             comm_r, comm_l, ctx_ref,
             send_sems_r, recv_sems_r, send_sems_l, recv_sems_l):
        my_pos = lax.axis_index("i")
        left = (my_pos - 1) % N_DEV
        right = (my_pos + 1) % N_DEV

        barrier_sem = pltpu.get_barrier_semaphore()
        for nbr in (left, right):
            pl.semaphore_signal(
                barrier_sem, inc=1,
                device_id=(nbr,), device_id_type=pl.DeviceIdType.MESH,
            )
        pl.semaphore_wait(barrier_sem, 2)

        qb = lax.broadcasted_iota(jnp.int32, (SQ, SKV), 0) // BLK
        kb = lax.broadcasted_iota(jnp.int32, (SQ, SKV), 1) // BLK
        mask = kb <= qb

        q_all = jnp.dot(x_ref[...], wq_ref[...],
                        preferred_element_type=jnp.float32)
        for b in range(B):
            for h in range(HQ_PER):
                q = q_all[b * SQ:(b + 1) * SQ, h * DH:(h + 1) * DH]
                s = lax.dot_general(
                    q, k_ref[b, h], (((1,), (1,)), ((), ())),
                    preferred_element_type=jnp.float32) * 0.125
                s = jnp.where(mask, s, -1e9)
                m = jnp.max(s, axis=-1, keepdims=True)
                w = jnp.exp(s - m)
                w = w / jnp.sum(w, axis=-1, keepdims=True)
                ctx_ref[b * SQ:(b + 1) * SQ, h * DH:(h + 1) * DH] = jnp.dot(
                    w, v_ref[b, h], preferred_element_type=jnp.float32)
        out_ref[...] = jnp.dot(ctx_ref[...], wo_ref[...],
                               preferred_element_type=jnp.float32)


        for t in range(N_DEV - 1):
            sc_r = (my_pos - t) % N_DEV
            rc_r = (my_pos - t - 1) % N_DEV
            sc_l = (my_pos + t) % N_DEV
            rc_l = (my_pos + t + 1) % N_DEV
            slot = t % 2
            rdma_r = pltpu.make_async_remote_copy(
                src_ref=out_ref.at[pl.ds(sc_r * CHUNK, CHUNK), :],
                dst_ref=comm_r.at[slot],
                send_sem=send_sems_r.at[slot],
                recv_sem=recv_sems_r.at[slot],
                device_id=(right,),
                device_id_type=pl.DeviceIdType.MESH,
            )
            rdma_l = pltpu.make_async_remote_copy(
                src_ref=out_ref.at[pl.ds(HALF + sc_l * CHUNK, CHUNK), :],
                dst_ref=comm_l.at[slot],
                send_sem=send_sems_l.at[slot],
                recv_sem=recv_sems_l.at[slot],
                device_id=(left,),
                device_id_type=pl.DeviceIdType.MESH,
            )
            rdma_r.start()
            rdma_l.start()
            rdma_r.wait()
            out_ref[pl.ds(rc_r * CHUNK, CHUNK), :] = (
                out_ref[pl.ds(rc_r * CHUNK, CHUNK), :] + comm_r[slot])
            rdma_l.wait()
            out_ref[pl.ds(HALF + rc_l * CHUNK, CHUNK), :] = (
                out_ref[pl.ds(HALF + rc_l * CHUNK, CHUNK), :] + comm_l[slot])

        for t in range(N_DEV - 1):
            ac_r = (my_pos + 1 - t) % N_DEV
            ac_l = (my_pos - 1 + t) % N_DEV
            slot = t % 2
            rdma_r = pltpu.make_async_remote_copy(
                src_ref=out_ref.at[pl.ds(ac_r * CHUNK, CHUNK), :],
                dst_ref=out_ref.at[pl.ds(ac_r * CHUNK, CHUNK), :],
                send_sem=send_sems_r.at[slot],
                recv_sem=recv_sems_r.at[slot],
                device_id=(right,),
                device_id_type=pl.DeviceIdType.MESH,
            )
            rdma_l = pltpu.make_async_remote_copy(
                src_ref=out_ref.at[pl.ds(HALF + ac_l * CHUNK, CHUNK), :],
                dst_ref=out_ref.at[pl.ds(HALF + ac_l * CHUNK, CHUNK), :],
                send_sem=send_sems_l.at[slot],
                recv_sem=recv_sems_l.at[slot],
                device_id=(left,),
                device_id_type=pl.DeviceIdType.MESH,
            )
            rdma_r.start()
            rdma_l.start()
            rdma_r.wait()
            rdma_l.wait()

    out2d = pl.pallas_call(
        body,
        out_shape=jax.ShapeDtypeStruct((ROWS, DM), jnp.float32),
        in_specs=[pl.BlockSpec(memory_space=pltpu.VMEM)] * 5,
        out_specs=pl.BlockSpec(memory_space=pltpu.VMEM),
        scratch_shapes=[
            pltpu.VMEM((2, CHUNK, DM), jnp.float32),
            pltpu.VMEM((2, CHUNK, DM), jnp.float32),
            pltpu.VMEM((ROWS, DP), jnp.float32),
            pltpu.SemaphoreType.DMA((2,)),
            pltpu.SemaphoreType.DMA((2,)),
            pltpu.SemaphoreType.DMA((2,)),
            pltpu.SemaphoreType.DMA((2,)),
        ],
        compiler_params=pltpu.CompilerParams(collective_id=0),
    )(xf, Wq, Kt, Vt, Wo)
    return out2d.reshape(B, SQ, DM)
